# Initial kernel scaffold; baseline (speedup 1.0000x reference)
import jax
import jax.numpy as jnp
from jax.experimental import pallas as pl


def kernel(candidates, lengths, batch, tgt_field, scores):
    raise NotImplementedError("write your pallas kernel here")


# speedup vs baseline: 3.1515x; 3.1515x over previous
import jax
import jax.numpy as jnp
from jax.experimental import pallas as pl
from jax.experimental.pallas import tpu as pltpu

PAD = 0
B, N, S = 128, 32, 2048
BLK = 8


def _rank_kernel(scores_ref, sorted_ref, best_ref):
    s = scores_ref[...]
    lane = jax.lax.broadcasted_iota(jnp.int32, (BLK, N), 1)
    rank = jnp.zeros((BLK, N), jnp.int32)
    for j in range(N):
        sj = s[:, j:j + 1]
        gt = (sj > s).astype(jnp.int32)
        tie = jnp.logical_and(sj == s, j < lane).astype(jnp.int32)
        rank = rank + gt + tie
    out = jnp.zeros((BLK, N), jnp.float32)
    for i in range(N):
        out = out + jnp.where(rank[:, i:i + 1] == lane, s[:, i:i + 1], 0.0)
    sorted_ref[...] = out
    best = jnp.sum(jnp.where(rank == 0, lane, 0), axis=1)
    best_ref[...] = best.reshape(1, 1, BLK)


def _gather_kernel(best_ref, cand_ref, out_ref, len_ref):
    row = cand_ref[...]
    out_ref[...] = row
    cnt = jnp.sum((row != PAD).astype(jnp.int32))
    len_ref[...] = jnp.full((1, 1, 128), cnt, jnp.int32)


def kernel(candidates, lengths, batch, tgt_field, scores):
    del lengths, batch, tgt_field
    nblk = B // BLK
    sorted_scores, best3 = pl.pallas_call(
        _rank_kernel,
        grid=(nblk,),
        in_specs=[pl.BlockSpec((BLK, N), lambda g: (g, 0))],
        out_specs=[
            pl.BlockSpec((BLK, N), lambda g: (g, 0)),
            pl.BlockSpec((1, 1, BLK), lambda g: (g, 0, 0)),
        ],
        out_shape=[
            jax.ShapeDtypeStruct((B, N), jnp.float32),
            jax.ShapeDtypeStruct((nblk, 1, BLK), jnp.int32),
        ],
    )(scores)
    best = best3.reshape(B)

    cand_flat = candidates.reshape(B * N, 1, S)
    grid_spec = pltpu.PrefetchScalarGridSpec(
        num_scalar_prefetch=1,
        grid=(B,),
        in_specs=[
            pl.BlockSpec((1, 1, S), lambda b, pref: (b * N + pref[b], 0, 0)),
        ],
        out_specs=[
            pl.BlockSpec((1, 1, S), lambda b, pref: (b, 0, 0)),
            pl.BlockSpec((1, 1, 128), lambda b, pref: (b, 0, 0)),
        ],
    )
    out3, len3 = pl.pallas_call(
        _gather_kernel,
        grid_spec=grid_spec,
        out_shape=[
            jax.ShapeDtypeStruct((B, 1, S), jnp.int32),
            jax.ShapeDtypeStruct((B, 1, 128), jnp.int32),
        ],
    )(best, cand_flat)
    output = out3.reshape(B, S)
    out_lengths = len3[:, 0, 0]
    return (output, out_lengths, sorted_scores)


# speedup vs baseline: 20.8149x; 6.6047x over previous
import functools

import jax
import jax.numpy as jnp
from jax import lax
from jax.experimental import pallas as pl
from jax.experimental.pallas import tpu as pltpu
from jax.experimental.pallas import tpu_sc as plsc

PAD = 0
B, N, S = 128, 32, 2048
NC, NS, L = 2, 16, 16
NW = NC * NS
BPW = B // NW

_mesh = plsc.VectorSubcoreMesh(core_axis_name="c", subcore_axis_name="s")


@functools.partial(
    pl.kernel,
    out_type=[
        jax.ShapeDtypeStruct((B, S), jnp.int32),
        jax.ShapeDtypeStruct((NW, L), jnp.int32),
        jax.ShapeDtypeStruct((B, 2, L), jnp.float32),
    ],
    mesh=_mesh,
    compiler_params=pltpu.CompilerParams(needs_layout_passes=False),
    scratch_types=[
        pltpu.VMEM((2, L), jnp.float32),
        pltpu.VMEM((2, L), jnp.float32),
        pltpu.VMEM((BPW, S), jnp.int32),
        pltpu.VMEM((L,), jnp.int32),
        pltpu.SemaphoreType.DMA,
    ],
)
def _sc_body(cand_hbm, scores_hbm, out_hbm, len_hbm, sorted_hbm,
             srow, sortrow, rows, lenv, sem):
    wid = lax.axis_index("s") * NC + lax.axis_index("c")
    i0 = lax.iota(jnp.int32, L)
    i1 = i0 + L
    z = jnp.zeros((L,), jnp.int32)

    row_copies = []
    for k in range(BPW):
        b = wid * BPW + k
        pltpu.sync_copy(scores_hbm.at[b], srow)
        s0 = srow[0]
        s1 = srow[1]

        def rank_step(j, carry, s0=s0, s1=s1):
            r0, r1 = carry
            bj = plsc.load_gather(
                srow,
                [jnp.full((L,), j // L, jnp.int32), jnp.full((L,), j % L, jnp.int32)],
            )
            hit0 = (bj > s0) | ((bj == s0) & (j < i0))
            hit1 = (bj > s1) | ((bj == s1) & (j < i1))
            return (r0 + jnp.where(hit0, 1, 0), r1 + jnp.where(hit1, 1, 0))

        r0, r1 = lax.fori_loop(0, N, rank_step, (z, z))
        plsc.store_scatter(sortrow, [r0 // L, r0 % L], s0)
        plsc.store_scatter(sortrow, [r1 // L, r1 % L], s1)
        pltpu.sync_copy(sortrow, sorted_hbm.at[b])

        best = jnp.sum(jnp.where(r0 == 0, i0, 0)) + jnp.sum(jnp.where(r1 == 0, i1, 0))
        row_copies.append(pltpu.async_copy(cand_hbm.at[b * N + best], rows.at[k], sem))

    cnts = z
    for k in range(BPW):
        row_copies[k].wait()

        def cnt_step(t, acc, k=k):
            chunk = rows[k, pl.ds(t * L, L)]
            return acc + plsc.all_reduce_population_count(chunk != PAD)

        c = lax.fori_loop(0, S // L, cnt_step, z)
        cnts = jnp.where(i0 == k, c, cnts)

    lenv[...] = cnts
    pltpu.sync_copy(lenv, len_hbm.at[wid])
    pltpu.sync_copy(rows, out_hbm.at[pl.ds(wid * BPW, BPW)])


def kernel(candidates, lengths, batch, tgt_field, scores):
    del lengths, batch, tgt_field
    cand_flat = candidates.reshape(B * N, S)
    scores2 = scores.reshape(B, 2, L)
    output, len_pad, sorted2 = _sc_body(cand_flat, scores2)
    out_lengths = len_pad[:, :BPW].reshape(B)
    sorted_scores = sorted2.reshape(B, N)
    return (output, out_lengths, sorted_scores)


# speedup vs baseline: 22.8609x; 1.0983x over previous
import functools

import jax
import jax.numpy as jnp
from jax import lax
from jax.experimental import pallas as pl
from jax.experimental.pallas import tpu as pltpu
from jax.experimental.pallas import tpu_sc as plsc

PAD = 0
B, N, S = 128, 32, 2048
NC, NS, L = 2, 16, 16
NW = NC * NS
BPW = B // NW

_mesh = plsc.VectorSubcoreMesh(core_axis_name="c", subcore_axis_name="s")


@functools.partial(
    pl.kernel,
    out_type=[
        jax.ShapeDtypeStruct((B, S), jnp.int32),
        jax.ShapeDtypeStruct((NW, L), jnp.int32),
        jax.ShapeDtypeStruct((NW, BPW, 2, L), jnp.float32),
    ],
    mesh=_mesh,
    compiler_params=pltpu.CompilerParams(needs_layout_passes=False),
    scratch_types=[
        pltpu.VMEM((BPW, 2, L), jnp.float32),
        pltpu.VMEM((BPW, 2, L), jnp.float32),
        pltpu.VMEM((BPW, S), jnp.int32),
        pltpu.VMEM((L,), jnp.int32),
        pltpu.SemaphoreType.DMA,
        pltpu.SemaphoreType.DMA,
    ],
)
def _sc_body(cand_hbm, scores_hbm, out_hbm, len_hbm, sorted_hbm,
             srows, sortrows, rows, lenv, sem, ssem):
    wid = lax.axis_index("s") * NC + lax.axis_index("c")
    i0 = lax.iota(jnp.int32, L)
    i1 = i0 + L
    z = jnp.zeros((L,), jnp.int32)

    pltpu.sync_copy(scores_hbm.at[wid], srows)

    row_copies = []
    for k in range(BPW):
        b = wid * BPW + k
        s0 = srows[k, 0]
        s1 = srows[k, 1]

        def rank_step(j, carry, k=k, s0=s0, s1=s1):
            r0, r1 = carry
            bj = plsc.load_gather(
                srows,
                [jnp.full((L,), k, jnp.int32), jnp.full((L,), j // L, jnp.int32),
                 jnp.full((L,), j % L, jnp.int32)],
            )
            hit0 = (bj > s0) | ((bj == s0) & (j < i0))
            hit1 = (bj > s1) | ((bj == s1) & (j < i1))
            return (r0 + jnp.where(hit0, 1, 0), r1 + jnp.where(hit1, 1, 0))

        r0, r1 = lax.fori_loop(0, N, rank_step, (z, z), unroll=4)

        best = jnp.sum(jnp.where(r0 == 0, i0, 0)) + jnp.sum(jnp.where(r1 == 0, i1, 0))
        row_copies.append(pltpu.async_copy(cand_hbm.at[b * N + best], rows.at[k], sem))

        kv = jnp.full((L,), k, jnp.int32)
        plsc.store_scatter(sortrows, [kv, r0 // L, r0 % L], s0)
        plsc.store_scatter(sortrows, [kv, r1 // L, r1 % L], s1)

    sorted_copy = pltpu.async_copy(sortrows, sorted_hbm.at[wid], ssem)

    cnts = z
    for k in range(BPW):
        row_copies[k].wait()

        def cnt_step(t, accs, k=k):
            new = []
            for u in range(4):
                chunk = rows[k, pl.ds((t * 4 + u) * L, L)]
                new.append(accs[u] + jnp.where(chunk != PAD, 1, 0))
            return tuple(new)

        accs = lax.fori_loop(0, S // L // 4, cnt_step, (z, z, z, z))
        c = jnp.sum(accs[0] + accs[1] + accs[2] + accs[3])
        cnts = jnp.where(i0 == k, c, cnts)

    lenv[...] = cnts
    pltpu.sync_copy(lenv, len_hbm.at[wid])
    pltpu.sync_copy(rows, out_hbm.at[pl.ds(wid * BPW, BPW)])
    sorted_copy.wait()


def kernel(candidates, lengths, batch, tgt_field, scores):
    del lengths, batch, tgt_field
    cand_flat = candidates.reshape(B * N, S)
    scores2 = scores.reshape(NW, BPW, 2, L)
    output, len_pad, sorted4 = _sc_body(cand_flat, scores2)
    out_lengths = len_pad[:, :BPW].reshape(B)
    sorted_scores = sorted4.reshape(B, N)
    return (output, out_lengths, sorted_scores)


# speedup vs baseline: 22.8703x; 1.0004x over previous
import functools

import jax
import jax.numpy as jnp
from jax import lax
from jax.experimental import pallas as pl
from jax.experimental.pallas import tpu as pltpu
from jax.experimental.pallas import tpu_sc as plsc

PAD = 0
B, N, S = 128, 32, 2048
NC, NS, L = 2, 16, 16
NW = NC * NS
BPW = B // NW

_mesh = plsc.VectorSubcoreMesh(core_axis_name="c", subcore_axis_name="s")


@functools.partial(
    pl.kernel,
    out_type=[
        jax.ShapeDtypeStruct((B, S), jnp.int32),
        jax.ShapeDtypeStruct((NW, L), jnp.int32),
        jax.ShapeDtypeStruct((NW, BPW, 2, L), jnp.float32),
    ],
    mesh=_mesh,
    compiler_params=pltpu.CompilerParams(
        needs_layout_passes=False,
        disable_bounds_checks=True,
        disable_semaphore_checks=True,
        skip_device_barrier=True,
    ),
    scratch_types=[
        pltpu.VMEM((BPW, 2, L), jnp.float32),
        pltpu.VMEM((BPW, 2, L), jnp.float32),
        pltpu.VMEM((BPW, S), jnp.int32),
        pltpu.VMEM((L,), jnp.int32),
        pltpu.SemaphoreType.DMA,
        pltpu.SemaphoreType.DMA,
    ],
)
def _sc_body(cand_hbm, scores_hbm, out_hbm, len_hbm, sorted_hbm,
             srows, sortrows, rows, lenv, sem, ssem):
    wid = lax.axis_index("s") * NC + lax.axis_index("c")
    i0 = lax.iota(jnp.int32, L)
    i1 = i0 + L
    z = jnp.zeros((L,), jnp.int32)

    pltpu.sync_copy(scores_hbm.at[wid], srows)

    row_copies = []
    for k in range(BPW):
        b = wid * BPW + k
        s0 = srows[k, 0]
        s1 = srows[k, 1]

        def rank_step(j, carry, k=k, s0=s0, s1=s1):
            r0, r1 = carry
            bj = plsc.load_gather(
                srows,
                [jnp.full((L,), k, jnp.int32), jnp.full((L,), j // L, jnp.int32),
                 jnp.full((L,), j % L, jnp.int32)],
            )
            hit0 = (bj > s0) | ((bj == s0) & (j < i0))
            hit1 = (bj > s1) | ((bj == s1) & (j < i1))
            return (r0 + jnp.where(hit0, 1, 0), r1 + jnp.where(hit1, 1, 0))

        r0, r1 = lax.fori_loop(0, N, rank_step, (z, z), unroll=4)

        best = jnp.sum(jnp.where(r0 == 0, i0, 0)) + jnp.sum(jnp.where(r1 == 0, i1, 0))
        row_copies.append(pltpu.async_copy(cand_hbm.at[b * N + best], rows.at[k], sem))

        kv = jnp.full((L,), k, jnp.int32)
        plsc.store_scatter(sortrows, [kv, r0 // L, r0 % L], s0)
        plsc.store_scatter(sortrows, [kv, r1 // L, r1 % L], s1)

    sorted_copy = pltpu.async_copy(sortrows, sorted_hbm.at[wid], ssem)

    cnts = z
    for k in range(BPW):
        row_copies[k].wait()

        def cnt_step(t, accs, k=k):
            new = []
            for u in range(4):
                chunk = rows[k, pl.ds((t * 4 + u) * L, L)]
                new.append(accs[u] + jnp.where(chunk != PAD, 1, 0))
            return tuple(new)

        accs = lax.fori_loop(0, S // L // 4, cnt_step, (z, z, z, z))
        c = jnp.sum(accs[0] + accs[1] + accs[2] + accs[3])
        cnts = jnp.where(i0 == k, c, cnts)

    lenv[...] = cnts
    pltpu.sync_copy(lenv, len_hbm.at[wid])
    pltpu.sync_copy(rows, out_hbm.at[pl.ds(wid * BPW, BPW)])
    sorted_copy.wait()


def kernel(candidates, lengths, batch, tgt_field, scores):
    del lengths, batch, tgt_field
    cand_flat = candidates.reshape(B * N, S)
    scores2 = scores.reshape(NW, BPW, 2, L)
    output, len_pad, sorted4 = _sc_body(cand_flat, scores2)
    out_lengths = len_pad[:, :BPW].reshape(B)
    sorted_scores = sorted4.reshape(B, N)
    return (output, out_lengths, sorted_scores)


# speedup vs baseline: 22.8911x; 1.0009x over previous
import functools

import jax
import jax.numpy as jnp
from jax import lax
from jax.experimental import pallas as pl
from jax.experimental.pallas import tpu as pltpu
from jax.experimental.pallas import tpu_sc as plsc

PAD = 0
B, N, S = 128, 32, 2048
NC, NS, L = 2, 16, 16
NW = NC * NS
BPW = B // NW

_mesh = plsc.VectorSubcoreMesh(core_axis_name="c", subcore_axis_name="s")


@functools.partial(
    pl.kernel,
    out_type=[
        jax.ShapeDtypeStruct((B, S), jnp.int32),
        jax.ShapeDtypeStruct((NW, L), jnp.int32),
        jax.ShapeDtypeStruct((NW, BPW, 2, L), jnp.float32),
    ],
    mesh=_mesh,
    compiler_params=pltpu.CompilerParams(needs_layout_passes=False),
    scratch_types=[
        pltpu.VMEM((BPW, 2, L), jnp.float32),
        pltpu.VMEM((BPW, 2, L), jnp.float32),
        pltpu.VMEM((BPW, S), jnp.int32),
        pltpu.VMEM((L,), jnp.int32),
        pltpu.SemaphoreType.DMA,
        pltpu.SemaphoreType.DMA,
    ],
)
def _sc_body(cand_hbm, scores_hbm, out_hbm, len_hbm, sorted_hbm,
             srows, sortrows, rows, lenv, sem, ssem):
    wid = lax.axis_index("s") * NC + lax.axis_index("c")
    i0 = lax.iota(jnp.int32, L)
    i1 = i0 + L
    z = jnp.zeros((L,), jnp.int32)

    pltpu.sync_copy(scores_hbm.at[wid], srows)

    row_copies = []
    for k in range(BPW):
        b = wid * BPW + k
        s0 = srows[k, 0]
        s1 = srows[k, 1]

        def rank_step(j, carry, k=k, s0=s0, s1=s1):
            r0, r1 = carry
            bj = plsc.load_gather(
                srows,
                [jnp.full((L,), k, jnp.int32), jnp.full((L,), j // L, jnp.int32),
                 jnp.full((L,), j % L, jnp.int32)],
            )
            hit0 = (bj > s0) | ((bj == s0) & (j < i0))
            hit1 = (bj > s1) | ((bj == s1) & (j < i1))
            return (r0 + jnp.where(hit0, 1, 0), r1 + jnp.where(hit1, 1, 0))

        r0, r1 = lax.fori_loop(0, N, rank_step, (z, z), unroll=4)

        best = jnp.sum(jnp.where(r0 == 0, i0, 0)) + jnp.sum(jnp.where(r1 == 0, i1, 0))
        row_copies.append(pltpu.async_copy(cand_hbm.at[b * N + best], rows.at[k], sem))

        kv = jnp.full((L,), k, jnp.int32)
        plsc.store_scatter(sortrows, [kv, r0 // L, r0 % L], s0)
        plsc.store_scatter(sortrows, [kv, r1 // L, r1 % L], s1)

    sorted_copy = pltpu.async_copy(sortrows, sorted_hbm.at[wid], ssem)

    cnts = z
    for k in range(BPW):
        row_copies[k].wait()

        def cnt_step(t, accs, k=k):
            new = []
            for u in range(4):
                chunk = rows[k, pl.ds((t * 4 + u) * L, L)]
                new.append(accs[u] + jnp.where(chunk != PAD, 1, 0))
            return tuple(new)

        accs = lax.fori_loop(0, S // L // 4, cnt_step, (z, z, z, z))
        c = jnp.sum(accs[0] + accs[1] + accs[2] + accs[3])
        cnts = jnp.where(i0 == k, c, cnts)

    lenv[...] = cnts
    pltpu.sync_copy(lenv, len_hbm.at[wid])
    pltpu.sync_copy(rows, out_hbm.at[pl.ds(wid * BPW, BPW)])
    sorted_copy.wait()


def kernel(candidates, lengths, batch, tgt_field, scores):
    del lengths, batch, tgt_field
    cand_flat = candidates.reshape(B * N, S)
    scores2 = scores.reshape(NW, BPW, 2, L)
    output, len_pad, sorted4 = _sc_body(cand_flat, scores2)
    out_lengths = len_pad[:, :BPW].reshape(B)
    sorted_scores = sorted4.reshape(B, N)
    return (output, out_lengths, sorted_scores)


# speedup vs baseline: 22.9159x; 1.0011x over previous
import functools

import jax
import jax.numpy as jnp
from jax import lax
from jax.experimental import pallas as pl
from jax.experimental.pallas import tpu as pltpu
from jax.experimental.pallas import tpu_sc as plsc

PAD = 0
B, N, S = 128, 32, 2048
NC, NS, L = 2, 16, 16
NW = NC * NS
BPW = B // NW

_mesh = plsc.VectorSubcoreMesh(core_axis_name="c", subcore_axis_name="s")


@functools.partial(
    pl.kernel,
    out_type=[
        jax.ShapeDtypeStruct((B, S), jnp.int32),
        jax.ShapeDtypeStruct((NW, L), jnp.int32),
        jax.ShapeDtypeStruct((NW, BPW, 2, L), jnp.float32),
    ],
    mesh=_mesh,
    compiler_params=pltpu.CompilerParams(needs_layout_passes=False),
    scratch_types=[
        pltpu.VMEM((BPW, 2, L), jnp.float32),
        pltpu.VMEM((BPW, 2, L), jnp.float32),
        pltpu.VMEM((BPW, S), jnp.int32),
        pltpu.VMEM((L,), jnp.int32),
        pltpu.SemaphoreType.DMA,
        pltpu.SemaphoreType.DMA,
    ],
)
def _sc_body(cand_hbm, scores_hbm, out_hbm, len_hbm, sorted_hbm,
             srows, sortrows, rows, lenv, sem, ssem):
    wid = lax.axis_index("s") * NC + lax.axis_index("c")
    i0 = lax.iota(jnp.int32, L)
    i1 = i0 + L
    z = jnp.zeros((L,), jnp.int32)

    pltpu.sync_copy(scores_hbm.at[wid], srows)

    row_copies = []
    for k in range(BPW):
        b = wid * BPW + k
        s0 = srows[k, 0]
        s1 = srows[k, 1]

        def rank_step(j, carry, k=k, s0=s0, s1=s1):
            r0, r1 = carry
            bj = plsc.load_gather(
                srows,
                [jnp.full((L,), k, jnp.int32), jnp.full((L,), j // L, jnp.int32),
                 jnp.full((L,), j % L, jnp.int32)],
            )
            hit0 = (bj > s0) | ((bj == s0) & (j < i0))
            hit1 = (bj > s1) | ((bj == s1) & (j < i1))
            return (r0 + jnp.where(hit0, 1, 0), r1 + jnp.where(hit1, 1, 0))

        r0, r1 = lax.fori_loop(0, N, rank_step, (z, z), unroll=4)

        best = jnp.sum(jnp.where(r0 == 0, i0, 0)) + jnp.sum(jnp.where(r1 == 0, i1, 0))
        row_copies.append(pltpu.async_copy(cand_hbm.at[b * N + best], rows.at[k], sem))

        kv = jnp.full((L,), k, jnp.int32)
        plsc.store_scatter(sortrows, [kv, r0 // L, r0 % L], s0)
        plsc.store_scatter(sortrows, [kv, r1 // L, r1 % L], s1)

    sorted_copy = pltpu.async_copy(sortrows, sorted_hbm.at[wid], ssem)

    for k in range(BPW):
        row_copies[k].wait()
    out_copy = pltpu.async_copy(rows, out_hbm.at[pl.ds(wid * BPW, BPW)], sem)

    cnts = z
    for k in range(BPW):
        def cnt_step(t, accs, k=k):
            new = []
            for u in range(4):
                chunk = rows[k, pl.ds((t * 4 + u) * L, L)]
                new.append(accs[u] + jnp.where(chunk != PAD, 1, 0))
            return tuple(new)

        accs = lax.fori_loop(0, S // L // 4, cnt_step, (z, z, z, z), unroll=2)
        c = jnp.sum(accs[0] + accs[1] + accs[2] + accs[3])
        cnts = jnp.where(i0 == k, c, cnts)

    lenv[...] = cnts
    pltpu.sync_copy(lenv, len_hbm.at[wid])
    out_copy.wait()
    sorted_copy.wait()


def kernel(candidates, lengths, batch, tgt_field, scores):
    del lengths, batch, tgt_field
    cand_flat = candidates.reshape(B * N, S)
    scores2 = scores.reshape(NW, BPW, 2, L)
    output, len_pad, sorted4 = _sc_body(cand_flat, scores2)
    out_lengths = len_pad[:, :BPW].reshape(B)
    sorted_scores = sorted4.reshape(B, N)
    return (output, out_lengths, sorted_scores)
